# Initial kernel scaffold; baseline (speedup 1.0000x reference)
import jax
import jax.numpy as jnp
from jax.experimental import pallas as pl


def kernel(h, edge_index):
    raise NotImplementedError("write your pallas kernel here")


# speedup vs baseline: 1.2003x; 1.2003x over previous
import jax
import jax.numpy as jnp
from jax import lax
from jax.experimental import pallas as pl
from jax.experimental.pallas import tpu as pltpu
from jax.experimental.pallas import tpu_sc as plsc

L = 16
NC, NS = 2, 16
NW = NC * NS
CHUNK = 400


def _dot_body(h_hbm, src_hbm, dst_hbm, out_hbm,
              idx_u, idx_v, u_rows, v_rows, out_v, sem_u, sem_v):
    e_total = out_hbm.shape[0]
    d = h_hbm.shape[1]
    c = idx_u.shape[0]
    epw = e_total // NW
    n_chunks = epw // c

    wid = lax.axis_index("s") * NC + lax.axis_index("c")
    wbase = wid * epw

    def chunk_body(ci, carry):
        base = wbase + ci * c
        pltpu.sync_copy(src_hbm.at[pl.ds(base, c)], idx_u)
        pltpu.sync_copy(dst_hbm.at[pl.ds(base, c)], idx_v)
        cp_u = pltpu.async_copy(h_hbm.at[idx_u], u_rows, sem_u)
        cp_v = pltpu.async_copy(h_hbm.at[idx_v], v_rows, sem_v)
        cp_u.wait()
        cp_v.wait()

        def group_body(g, gcarry):
            rows = g * L + lax.iota(jnp.int32, L)
            acc = jnp.zeros((L,), jnp.float32)
            for k in range(d):
                col = jnp.full((L,), k, jnp.int32)
                uu = plsc.load_gather(u_rows, [rows, col])
                vv = plsc.load_gather(v_rows, [rows, col])
                acc = acc + uu * vv
            out_v[pl.ds(g * L, L)] = acc
            return gcarry

        lax.fori_loop(0, c // L, group_body, 0)
        pltpu.sync_copy(out_v, out_hbm.at[pl.ds(base, c)])
        return carry

    lax.fori_loop(0, n_chunks, chunk_body, 0)


def kernel(h, edge_index):
    e_total = edge_index.shape[1]
    d = h.shape[1]
    src = edge_index[0].astype(jnp.int32)
    dst = edge_index[1].astype(jnp.int32)

    sc_call = pl.kernel(
        _dot_body,
        out_type=jax.ShapeDtypeStruct((e_total,), jnp.float32),
        mesh=plsc.VectorSubcoreMesh(core_axis_name="c", subcore_axis_name="s"),
        scratch_types=[
            pltpu.VMEM((CHUNK,), jnp.int32),
            pltpu.VMEM((CHUNK,), jnp.int32),
            pltpu.VMEM((CHUNK, d), jnp.float32),
            pltpu.VMEM((CHUNK, d), jnp.float32),
            pltpu.VMEM((CHUNK,), jnp.float32),
            pltpu.SemaphoreType.DMA,
            pltpu.SemaphoreType.DMA,
        ],
        compiler_params=pltpu.CompilerParams(needs_layout_passes=False),
    )
    out = sc_call(h, src, dst)
    return out.reshape(e_total, 1)
